# Initial kernel scaffold; baseline (speedup 1.0000x reference)
import functools

import jax
import jax.numpy as jnp
from jax import lax
from jax.experimental import pallas as pl
from jax.experimental.pallas import tpu as pltpu
from jax.experimental.pallas import tpu_sc as plsc

B = 8
S = 2048
D = 2048
DK = 256
NODE = 64
L = 64
N = B * NODE
R = N * L

NC = 2
NS = 16
NW = NC * NS
LANES = 16

ROWS_PER_W = R // NW
BAGS_PER_W = N // NW
GCHUNK = 128


SBLK = 512


def _proj_body(x_ref, w_ref, b_ref, h_ref, wq_ref, p_ref, t_ref):
    x = x_ref[0]
    p = jnp.dot(x, w_ref[...], preferred_element_type=jnp.float32)
    p_ref[0] = p + b_ref[...]
    t_ref[...] = jnp.dot(h_ref[...], wq_ref[...],
                         preferred_element_type=jnp.float32)


def _projection(seq_output, W_pre, b_pre2, hidden2, W_q):
    return pl.pallas_call(
        _proj_body,
        grid=(B, S // SBLK),
        in_specs=[
            pl.BlockSpec((1, SBLK, D), lambda b, s: (b, s, 0)),
            pl.BlockSpec((D, DK), lambda b, s: (0, 0)),
            pl.BlockSpec((1, DK), lambda b, s: (0, 0)),
            pl.BlockSpec((1, D), lambda b, s: (b, 0)),
            pl.BlockSpec((D, DK), lambda b, s: (0, 0)),
        ],
        out_specs=[
            pl.BlockSpec((1, SBLK, DK), lambda b, s: (b, s, 0)),
            pl.BlockSpec((1, DK), lambda b, s: (b, 0)),
        ],
        out_shape=[
            jax.ShapeDtypeStruct((B, S, DK), jnp.float32),
            jax.ShapeDtypeStruct((B, DK), jnp.float32),
        ],
    )(seq_output, W_pre, b_pre2, hidden2, W_q)


def _sc_gather_body(p2, idxf, g_out, idx_v, buf, sem):
    wid = lax.axis_index("s") * NC + lax.axis_index("c")
    base = wid * ROWS_PER_W
    pltpu.sync_copy(idxf.at[pl.ds(base, ROWS_PER_W)], idx_v)
    off = (wid // (NODE * L // ROWS_PER_W)) * S
    offv = jnp.full((LANES,), 0, jnp.int32) + off
    for j in range(ROWS_PER_W // LANES):
        sl = pl.ds(j * LANES, LANES)
        idx_v[sl] = idx_v[sl] + offv
    for c in range(ROWS_PER_W // GCHUNK):
        pltpu.async_copy(
            p2.at[idx_v.at[pl.ds(c * GCHUNK, GCHUNK)]], buf, sem).wait()
        pltpu.sync_copy(buf, g_out.at[pl.ds(base + c * GCHUNK, GCHUNK)])


def _sc_gather(P2, idx_flat):
    mesh = plsc.VectorSubcoreMesh(
        core_axis_name="c", subcore_axis_name="s",
        num_cores=NC, num_subcores=NS)
    k = pl.kernel(
        _sc_gather_body,
        out_type=jax.ShapeDtypeStruct((R, DK), jnp.float32),
        mesh=mesh,
        scratch_types=[
            pltpu.VMEM((ROWS_PER_W,), jnp.int32),
            pltpu.VMEM((GCHUNK, DK), jnp.float32),
            pltpu.SemaphoreType.DMA,
        ],
    )
    return k(P2, idx_flat)


def _attn_body(g_ref, t_ref, wv_ref, s_ref):
    g = g_ref[...]
    t = t_ref[0]
    wv = wv_ref[0]
    tmp = jnp.tanh(g + t[None, None, :])
    e = jnp.sum(tmp * wv[None, None, :], axis=2)
    m = jnp.max(e, axis=1, keepdims=True)
    p = jnp.exp(e - m)
    s_ref[...] = p / jnp.sum(p, axis=1, keepdims=True)


def _attention_scores(G3, target, wv2):
    return pl.pallas_call(
        _attn_body,
        grid=(B,),
        in_specs=[
            pl.BlockSpec((NODE, L, DK), lambda b: (b, 0, 0)),
            pl.BlockSpec((1, DK), lambda b: (b, 0)),
            pl.BlockSpec((1, DK), lambda b: (0, 0)),
        ],
        out_specs=pl.BlockSpec((NODE, L), lambda b: (b, 0)),
        out_shape=jax.ShapeDtypeStruct((N, L), jnp.float32),
    )(G3, target, wv2)


def _sc_scatter_body(score, idx2, a_out, sc_v, ix_v, acc):
    wid = lax.axis_index("s") * NC + lax.axis_index("c")
    base = wid * BAGS_PER_W
    pltpu.sync_copy(score.at[pl.ds(base, BAGS_PER_W)], sc_v)
    pltpu.sync_copy(idx2.at[pl.ds(base, BAGS_PER_W)], ix_v)
    zeros = jnp.zeros((LANES,), jnp.float32)

    def zero_row(j, _):
        for i in range(BAGS_PER_W):
            acc[i, pl.ds(j * LANES, LANES)] = zeros
        return 0

    lax.fori_loop(0, S // LANES, zero_row, 0)
    rows = lax.iota(jnp.int32, LANES)
    for l in range(L):
        col = jnp.full((LANES,), l, jnp.int32)
        iv = plsc.load_gather(ix_v, [rows, col])
        sv = plsc.load_gather(sc_v, [rows, col])
        plsc.addupdate_scatter(acc, [rows, iv], sv)
    pltpu.sync_copy(acc, a_out.at[pl.ds(base, BAGS_PER_W)])


def _sc_scatter(score, idx2):
    mesh = plsc.VectorSubcoreMesh(
        core_axis_name="c", subcore_axis_name="s",
        num_cores=NC, num_subcores=NS)
    k = pl.kernel(
        _sc_scatter_body,
        out_type=jax.ShapeDtypeStruct((N, S), jnp.float32),
        mesh=mesh,
        scratch_types=[
            pltpu.VMEM((BAGS_PER_W, L), jnp.float32),
            pltpu.VMEM((BAGS_PER_W, L), jnp.int32),
            pltpu.VMEM((BAGS_PER_W, S), jnp.float32),
        ],
    )
    return k(score, idx2)


def _ctx_body(a_ref, x_ref, nl_ref, n_ref, m_ref):
    a = a_ref[0]
    x = x_ref[0]
    n_ref[0] = jnp.dot(a, x, preferred_element_type=jnp.float32)
    pos = lax.broadcasted_iota(jnp.int32, (1, NODE), 1)
    m_ref[...] = (pos < nl_ref[0]).astype(jnp.float32)


def _context(A3, seq_output, node_lengths):
    return pl.pallas_call(
        _ctx_body,
        grid=(B,),
        in_specs=[
            pl.BlockSpec((1, NODE, S), lambda b: (b, 0, 0)),
            pl.BlockSpec((1, S, D), lambda b: (b, 0, 0)),
            pl.BlockSpec(memory_space=pltpu.SMEM),
        ],
        out_specs=[
            pl.BlockSpec((1, NODE, D), lambda b: (b, 0, 0)),
            pl.BlockSpec((1, NODE), lambda b: (b, 0)),
        ],
        out_shape=[
            jax.ShapeDtypeStruct((B, NODE, D), jnp.float32),
            jax.ShapeDtypeStruct((B, NODE), jnp.float32),
        ],
    )(A3, seq_output, node_lengths)


def kernel(seq_output, hidden, index, lengths, node_lengths, feat_seqs,
           node_type, W_pre, b_pre, W_q, W_v, max_length):
    hidden2 = jnp.transpose(hidden, (1, 0, 2)).reshape(B, D)
    P, target = _projection(seq_output, W_pre, b_pre.reshape(1, DK),
                            hidden2, W_q)
    P2 = P.reshape(B * S, DK)
    idx_flat = index.reshape(R).astype(jnp.int32)
    G = _sc_gather(P2, idx_flat)
    G3 = G.reshape(N, L, DK)
    score = _attention_scores(G3, target, W_v.reshape(1, DK))
    idx2 = index.reshape(N, L).astype(jnp.int32)
    A = _sc_scatter(score, idx2)
    A3 = A.reshape(B, NODE, S)
    nodes, nodes_mask = _context(A3, seq_output, node_lengths)
    return nodes, nodes_mask, hidden2


# speedup vs baseline: 5.3080x; 5.3080x over previous
import functools

import jax
import jax.numpy as jnp
from jax import lax
from jax.experimental import pallas as pl
from jax.experimental.pallas import tpu as pltpu
from jax.experimental.pallas import tpu_sc as plsc

B = 8
S = 2048
D = 2048
DK = 256
NODE = 64
L = 64
N = B * NODE
R = N * L

NC = 2
NS = 16
NW = NC * NS
LANES = 16

ROWS_PER_W = R // NW
BAGS_PER_W = N // NW
GCHUNK = 128


SBLK = 512


def _proj_body(x_ref, w_ref, b_ref, p_ref):
    x = x_ref[0]
    p = jnp.dot(x, w_ref[...], preferred_element_type=jnp.float32)
    p_ref[0] = p + b_ref[...]


def _projection(seq_output, W_pre, b_pre2):
    return pl.pallas_call(
        _proj_body,
        grid=(B, S // SBLK),
        in_specs=[
            pl.BlockSpec((1, SBLK, D), lambda b, s: (b, s, 0)),
            pl.BlockSpec((D, DK), lambda b, s: (0, 0)),
            pl.BlockSpec((1, DK), lambda b, s: (0, 0)),
        ],
        out_specs=pl.BlockSpec((1, SBLK, DK), lambda b, s: (b, s, 0)),
        out_shape=jax.ShapeDtypeStruct((B, S, DK), jnp.float32),
    )(seq_output, W_pre, b_pre2)


def _sc_gather_body(p2, idxf, g_out, idx_v, buf, sem):
    wid = lax.axis_index("s") * NC + lax.axis_index("c")
    base = wid * ROWS_PER_W
    pltpu.sync_copy(idxf.at[pl.ds(base, ROWS_PER_W)], idx_v)
    off = (wid // (NODE * L // ROWS_PER_W)) * S
    offv = jnp.full((LANES,), 0, jnp.int32) + off
    for j in range(ROWS_PER_W // LANES):
        sl = pl.ds(j * LANES, LANES)
        idx_v[sl] = idx_v[sl] + offv
    for c in range(ROWS_PER_W // GCHUNK):
        pltpu.async_copy(
            p2.at[idx_v.at[pl.ds(c * GCHUNK, GCHUNK)]], buf, sem).wait()
        pltpu.sync_copy(buf, g_out.at[pl.ds(base + c * GCHUNK, GCHUNK)])


def _sc_gather(P2, idx_flat):
    mesh = plsc.VectorSubcoreMesh(
        core_axis_name="c", subcore_axis_name="s",
        num_cores=NC, num_subcores=NS)
    k = pl.kernel(
        _sc_gather_body,
        out_type=jax.ShapeDtypeStruct((R, DK), jnp.float32),
        mesh=mesh,
        scratch_types=[
            pltpu.VMEM((ROWS_PER_W,), jnp.int32),
            pltpu.VMEM((GCHUNK, DK), jnp.float32),
            pltpu.SemaphoreType.DMA,
        ],
    )
    return k(P2, idx_flat)


def _attn_body(g_ref, h_ref, wq_ref, wv_ref, s_ref):
    g = g_ref[...]
    t = jnp.dot(h_ref[0], wq_ref[...],
                preferred_element_type=jnp.float32)
    wv = wv_ref[0]
    tmp = jnp.tanh(g + t[0][None, None, :])
    e = jnp.sum(tmp * wv[None, None, :], axis=2)
    m = jnp.max(e, axis=1, keepdims=True)
    p = jnp.exp(e - m)
    s_ref[...] = p / jnp.sum(p, axis=1, keepdims=True)


def _attention_scores(G3, hidden3, W_q, wv2):
    return pl.pallas_call(
        _attn_body,
        grid=(B,),
        in_specs=[
            pl.BlockSpec((NODE, L, DK), lambda b: (b, 0, 0)),
            pl.BlockSpec((1, 1, D), lambda b: (b, 0, 0)),
            pl.BlockSpec((D, DK), lambda b: (0, 0)),
            pl.BlockSpec((1, DK), lambda b: (0, 0)),
        ],
        out_specs=pl.BlockSpec((NODE, L), lambda b: (b, 0)),
        out_shape=jax.ShapeDtypeStruct((N, L), jnp.float32),
    )(G3, hidden3, W_q, wv2)


def _sc_scatter_body(score_t, idx_t, a_out, sc_v, ix_v, acc):
    wid = lax.axis_index("s") * NC + lax.axis_index("c")
    base = wid * BAGS_PER_W
    pltpu.sync_copy(score_t.at[wid], sc_v)
    pltpu.sync_copy(idx_t.at[wid], ix_v)
    zeros = jnp.zeros((LANES,), jnp.float32)

    def zero_row(j, _):
        for i in range(BAGS_PER_W):
            acc[i, pl.ds(j * LANES, LANES)] = zeros
        return 0

    lax.fori_loop(0, S // LANES, zero_row, 0)
    rows = lax.iota(jnp.int32, LANES)
    for l in range(L):
        iv = ix_v[l, :]
        sv = sc_v[l, :]
        plsc.addupdate_scatter(acc, [rows, iv], sv)
    pltpu.sync_copy(acc, a_out.at[pl.ds(base, BAGS_PER_W)])


def _sc_scatter(score_t, idx_t):
    mesh = plsc.VectorSubcoreMesh(
        core_axis_name="c", subcore_axis_name="s",
        num_cores=NC, num_subcores=NS)
    k = pl.kernel(
        _sc_scatter_body,
        out_type=jax.ShapeDtypeStruct((N, S), jnp.float32),
        mesh=mesh,
        scratch_types=[
            pltpu.VMEM((L, BAGS_PER_W), jnp.float32),
            pltpu.VMEM((L, BAGS_PER_W), jnp.int32),
            pltpu.VMEM((BAGS_PER_W, S), jnp.float32),
        ],
        compiler_params=pltpu.CompilerParams(use_tc_tiling_on_sc=False,
                                             needs_layout_passes=False),
    )
    return k(score_t, idx_t)


def _ctx_body(a_ref, x_ref, nl_ref, n_ref, m_ref):
    a = a_ref[0]
    x = x_ref[0]
    n_ref[0] = jnp.dot(a, x, preferred_element_type=jnp.float32)
    pos = lax.broadcasted_iota(jnp.int32, (1, 1, NODE), 2)
    m_ref[...] = (pos < nl_ref[0]).astype(jnp.float32)


def _context(A3, seq_output, node_lengths):
    return pl.pallas_call(
        _ctx_body,
        grid=(B,),
        in_specs=[
            pl.BlockSpec((1, NODE, S), lambda b: (b, 0, 0)),
            pl.BlockSpec((1, S, D), lambda b: (b, 0, 0)),
            pl.BlockSpec(memory_space=pltpu.SMEM),
        ],
        out_specs=[
            pl.BlockSpec((1, NODE, D), lambda b: (b, 0, 0)),
            pl.BlockSpec((1, 1, NODE), lambda b: (b, 0, 0)),
        ],
        out_shape=[
            jax.ShapeDtypeStruct((B, NODE, D), jnp.float32),
            jax.ShapeDtypeStruct((B, 1, NODE), jnp.float32),
        ],
    )(A3, seq_output, node_lengths)


def kernel(seq_output, hidden, index, lengths, node_lengths, feat_seqs,
           node_type, W_pre, b_pre, W_q, W_v, max_length):
    hidden2 = jnp.transpose(hidden, (1, 0, 2)).reshape(B, D)
    P = _projection(seq_output, W_pre, b_pre.reshape(1, DK))
    P2 = P.reshape(B * S, DK)
    idx_flat = index.reshape(R).astype(jnp.int32)
    G = _sc_gather(P2, idx_flat)
    G3 = G.reshape(N, L, DK)
    score = _attention_scores(G3, hidden2.reshape(B, 1, D), W_q,
                              W_v.reshape(1, DK))
    idx2 = index.reshape(N, L).astype(jnp.int32)
    score3 = score.reshape(NW, BAGS_PER_W, L).transpose(0, 2, 1)
    idx3 = idx2.reshape(NW, BAGS_PER_W, L).transpose(0, 2, 1)
    A = _sc_scatter(score3, idx3)
    A3 = A.reshape(B, NODE, S)
    nodes, nodes_mask3 = _context(A3, seq_output, node_lengths)
    return nodes, nodes_mask3.reshape(B, NODE), hidden2
